# baseline (speedup 1.0000x reference)
import functools

import jax
import jax.numpy as jnp
from jax import lax
from jax.experimental import pallas as pl
from jax.experimental.pallas import tpu as pltpu
from jax.experimental.pallas import tpu_sc as plsc

B = 32
S = 512
HIDDEN = 768
EPS = 1e-12
CHUNK = 64
NW = 32
BH = B // 2


def _make_sc_gather(n_rows):
    n_tok = n_rows * S
    per_worker = n_tok // NW
    nchunk = per_worker // CHUNK

    def body(ids_hbm, word_hbm, out_hbm, ids_v, buf0, buf1,
             sem_g0, sem_g1, sem_o0, sem_o1):
        wid = lax.axis_index("s") * 2 + lax.axis_index("c")
        row_base = pl.multiple_of(wid * per_worker, per_worker)
        pltpu.sync_copy(ids_hbm.at[pl.ds(row_base, per_worker)], ids_v)

        bufs = (buf0, buf1)
        gsems = (sem_g0, sem_g1)
        osems = (sem_o0, sem_o1)

        def start_gather(c):
            p = c % 2
            return pltpu.async_copy(
                word_hbm.at[ids_v.at[pl.ds(c * CHUNK, CHUNK)]],
                bufs[p], gsems[p])

        def start_out(c):
            p = c % 2
            return pltpu.async_copy(
                bufs[p], out_hbm.at[pl.ds(row_base + c * CHUNK, CHUNK)],
                osems[p])

        out_h = {}
        gather_h = {0: start_gather(0)}
        for c in range(nchunk):
            if c + 1 < nchunk:
                if c - 1 >= 0:
                    out_h[c - 1].wait()
                gather_h[c + 1] = start_gather(c + 1)
            gather_h[c].wait()
            out_h[c] = start_out(c)
        for c in range(max(0, nchunk - 2), nchunk):
            out_h[c].wait()

    return functools.partial(
        pl.kernel,
        mesh=plsc.VectorSubcoreMesh(core_axis_name="c", subcore_axis_name="s"),
        out_type=jax.ShapeDtypeStruct((n_tok, HIDDEN), jnp.float32),
        scratch_types=[
            pltpu.VMEM((per_worker,), jnp.int32),
            pltpu.VMEM((CHUNK, HIDDEN), jnp.float32),
            pltpu.VMEM((CHUNK, HIDDEN), jnp.float32),
            pltpu.SemaphoreType.DMA,
            pltpu.SemaphoreType.DMA,
            pltpu.SemaphoreType.DMA,
            pltpu.SemaphoreType.DMA,
        ],
    )(body)


PIECES = (4, 8, 8, 12)


def _ln_block(gath_ref, pp_ref, tf_ref, diff_ref, g_ref, b_ref):
    x = (gath_ref[0] + pp_ref[...]
         + tf_ref[0, 0, :][:, None] * diff_ref[0][None, :])
    mean = jnp.mean(x, axis=-1, keepdims=True)
    var = jnp.mean(x * x, axis=-1, keepdims=True) - mean * mean
    inv = lax.rsqrt(var + EPS)
    return (x - mean) * inv * g_ref[0][None, :] + b_ref[0][None, :]


def _tc_first_body(gath_ref, pp_ref, tf_ref, diff_ref, g_ref, b_ref, o_ref):
    o_ref[0] = _ln_block(gath_ref, pp_ref, tf_ref, diff_ref, g_ref, b_ref)


def _tc_next_body(prev_ref, gath_ref, pp_ref, tf_ref, diff_ref, g_ref,
                  b_ref, o_ref):
    del prev_ref
    o_ref[0] = _ln_block(gath_ref, pp_ref, tf_ref, diff_ref, g_ref, b_ref)


_SMALL_SPECS = [
    pl.BlockSpec((S, HIDDEN), lambda b: (0, 0)),
    pl.BlockSpec((1, 1, S), lambda b: (b, 0, 0)),
    pl.BlockSpec((1, HIDDEN), lambda b: (0, 0)),
    pl.BlockSpec((1, HIDDEN), lambda b: (0, 0)),
    pl.BlockSpec((1, HIDDEN), lambda b: (0, 0)),
]


def _make_tc_ln(n_rows, row_off):
    first = row_off == 0
    body = _tc_first_body if first else _tc_next_body
    gspec = [pl.BlockSpec((1, S, HIDDEN), lambda b: (b, 0, 0))]
    if not first:
        gspec = [pl.BlockSpec(memory_space=pl.ANY)] + gspec
    return pl.pallas_call(
        body,
        grid=(n_rows,),
        in_specs=gspec + _SMALL_SPECS,
        out_specs=pl.BlockSpec(
            (1, S, HIDDEN), lambda b: (b + row_off, 0, 0)),
        out_shape=jax.ShapeDtypeStruct((B, S, HIDDEN), jnp.float32),
        input_output_aliases={} if first else {0: 0},
    )


_sc_gathers = [_make_sc_gather(n) for n in PIECES]
_tc_lns = []
_off = 0
for _n in PIECES:
    _tc_lns.append(_make_tc_ln(_n, _off))
    _off += _n


def kernel(input_ids, token_type_ids, word_table, pos_table, type_table,
           name_table, gamma, beta):
    ids = input_ids.astype(jnp.int32).reshape(B * S)
    pos_plus = pos_table + name_table[0][None, :] + type_table[0][None, :]
    diff = (type_table[1] - type_table[0])[None, :]
    tf = token_type_ids.astype(jnp.float32).reshape(B, 1, S)
    gamma2 = gamma[None, :]
    beta2 = beta[None, :]

    gs = []
    off = 0
    for i, n in enumerate(PIECES):
        gs.append(_sc_gathers[i](
            ids[off * S:(off + n) * S], word_table).reshape(n, S, HIDDEN))
        off += n

    out = None
    off = 0
    for i, n in enumerate(PIECES):
        args = (gs[i], pos_plus, tf[off:off + n], diff, gamma2, beta2)
        if i == 0:
            out = _tc_lns[i](*args)
        else:
            out = _tc_lns[i](out, *args)
        off += n
    return out


# speedup vs baseline: 1.0247x; 1.0247x over previous
import functools

import jax
import jax.numpy as jnp
from jax import lax
from jax.experimental import pallas as pl
from jax.experimental.pallas import tpu as pltpu
from jax.experimental.pallas import tpu_sc as plsc

B = 32
S = 512
HIDDEN = 768
EPS = 1e-12
CHUNK = 64
NW = 32
BH = B // 2


def _make_sc_gather(n_rows):
    n_tok = n_rows * S
    per_worker = n_tok // NW
    nchunk = per_worker // CHUNK

    def body(ids_hbm, word_hbm, out_hbm, ids_v, buf0, buf1,
             sem_g0, sem_g1, sem_o0, sem_o1):
        wid = lax.axis_index("s") * 2 + lax.axis_index("c")
        row_base = pl.multiple_of(wid * per_worker, per_worker)
        pltpu.sync_copy(ids_hbm.at[pl.ds(row_base, per_worker)], ids_v)

        bufs = (buf0, buf1)
        gsems = (sem_g0, sem_g1)
        osems = (sem_o0, sem_o1)

        def start_gather(c):
            p = c % 2
            return pltpu.async_copy(
                word_hbm.at[ids_v.at[pl.ds(c * CHUNK, CHUNK)]],
                bufs[p], gsems[p])

        def start_out(c):
            p = c % 2
            return pltpu.async_copy(
                bufs[p], out_hbm.at[pl.ds(row_base + c * CHUNK, CHUNK)],
                osems[p])

        out_h = {}
        gather_h = {0: start_gather(0)}
        for c in range(nchunk):
            if c + 1 < nchunk:
                if c - 1 >= 0:
                    out_h[c - 1].wait()
                gather_h[c + 1] = start_gather(c + 1)
            gather_h[c].wait()
            out_h[c] = start_out(c)
        for c in range(max(0, nchunk - 2), nchunk):
            out_h[c].wait()

    return functools.partial(
        pl.kernel,
        mesh=plsc.VectorSubcoreMesh(core_axis_name="c", subcore_axis_name="s"),
        out_type=jax.ShapeDtypeStruct((n_tok, HIDDEN), jnp.float32),
        scratch_types=[
            pltpu.VMEM((per_worker,), jnp.int32),
            pltpu.VMEM((CHUNK, HIDDEN), jnp.float32),
            pltpu.VMEM((CHUNK, HIDDEN), jnp.float32),
            pltpu.SemaphoreType.DMA,
            pltpu.SemaphoreType.DMA,
            pltpu.SemaphoreType.DMA,
            pltpu.SemaphoreType.DMA,
        ],
    )(body)


PIECES = (12, 20)


def _ln_block(gath_ref, pp_ref, tf_ref, diff_ref, g_ref, b_ref):
    x = (gath_ref[0] + pp_ref[...]
         + tf_ref[0, 0, :][:, None] * diff_ref[0][None, :])
    mean = jnp.mean(x, axis=-1, keepdims=True)
    var = jnp.mean(x * x, axis=-1, keepdims=True) - mean * mean
    inv = lax.rsqrt(var + EPS)
    return (x - mean) * inv * g_ref[0][None, :] + b_ref[0][None, :]


def _tc_first_body(gath_ref, pp_ref, tf_ref, diff_ref, g_ref, b_ref, o_ref):
    o_ref[0] = _ln_block(gath_ref, pp_ref, tf_ref, diff_ref, g_ref, b_ref)


def _tc_next_body(prev_ref, gath_ref, pp_ref, tf_ref, diff_ref, g_ref,
                  b_ref, o_ref):
    del prev_ref
    o_ref[0] = _ln_block(gath_ref, pp_ref, tf_ref, diff_ref, g_ref, b_ref)


_SMALL_SPECS = [
    pl.BlockSpec((S, HIDDEN), lambda b: (0, 0)),
    pl.BlockSpec((1, 1, S), lambda b: (b, 0, 0)),
    pl.BlockSpec((1, HIDDEN), lambda b: (0, 0)),
    pl.BlockSpec((1, HIDDEN), lambda b: (0, 0)),
    pl.BlockSpec((1, HIDDEN), lambda b: (0, 0)),
]


def _make_tc_ln(n_rows, row_off):
    first = row_off == 0
    body = _tc_first_body if first else _tc_next_body
    gspec = [pl.BlockSpec((1, S, HIDDEN), lambda b: (b, 0, 0))]
    if not first:
        gspec = [pl.BlockSpec(memory_space=pl.ANY)] + gspec
    return pl.pallas_call(
        body,
        grid=(n_rows,),
        in_specs=gspec + _SMALL_SPECS,
        out_specs=pl.BlockSpec(
            (1, S, HIDDEN), lambda b: (b + row_off, 0, 0)),
        out_shape=jax.ShapeDtypeStruct((B, S, HIDDEN), jnp.float32),
        input_output_aliases={} if first else {0: 0},
    )


_sc_gathers = [_make_sc_gather(n) for n in PIECES]
_tc_lns = []
_off = 0
for _n in PIECES:
    _tc_lns.append(_make_tc_ln(_n, _off))
    _off += _n


def kernel(input_ids, token_type_ids, word_table, pos_table, type_table,
           name_table, gamma, beta):
    ids = input_ids.astype(jnp.int32).reshape(B * S)
    pos_plus = pos_table + name_table[0][None, :] + type_table[0][None, :]
    diff = (type_table[1] - type_table[0])[None, :]
    tf = token_type_ids.astype(jnp.float32).reshape(B, 1, S)
    gamma2 = gamma[None, :]
    beta2 = beta[None, :]

    gs = []
    off = 0
    for i, n in enumerate(PIECES):
        gs.append(_sc_gathers[i](
            ids[off * S:(off + n) * S], word_table).reshape(n, S, HIDDEN))
        off += n

    out = None
    off = 0
    for i, n in enumerate(PIECES):
        args = (gs[i], pos_plus, tf[off:off + n], diff, gamma2, beta2)
        if i == 0:
            out = _tc_lns[i](*args)
        else:
            out = _tc_lns[i](out, *args)
        off += n
    return out


# speedup vs baseline: 1.0519x; 1.0266x over previous
import functools

import jax
import jax.numpy as jnp
from jax import lax
from jax.experimental import pallas as pl
from jax.experimental.pallas import tpu as pltpu
from jax.experimental.pallas import tpu_sc as plsc

B = 32
S = 512
HIDDEN = 768
EPS = 1e-12
CHUNK = 64
NW = 32
BH = B // 2


def _make_sc_gather(n_rows):
    n_tok = n_rows * S
    per_worker = n_tok // NW
    nchunk = per_worker // CHUNK

    def body(ids_hbm, word_hbm, out_hbm, ids_v, buf0, buf1,
             sem_g0, sem_g1, sem_o0, sem_o1):
        wid = lax.axis_index("s") * 2 + lax.axis_index("c")
        row_base = pl.multiple_of(wid * per_worker, per_worker)
        pltpu.sync_copy(ids_hbm.at[pl.ds(row_base, per_worker)], ids_v)

        bufs = (buf0, buf1)
        gsems = (sem_g0, sem_g1)
        osems = (sem_o0, sem_o1)

        def start_gather(c):
            p = c % 2
            return pltpu.async_copy(
                word_hbm.at[ids_v.at[pl.ds(c * CHUNK, CHUNK)]],
                bufs[p], gsems[p])

        def start_out(c):
            p = c % 2
            return pltpu.async_copy(
                bufs[p], out_hbm.at[pl.ds(row_base + c * CHUNK, CHUNK)],
                osems[p])

        out_h = {}
        gather_h = {0: start_gather(0)}
        for c in range(nchunk):
            if c + 1 < nchunk:
                if c - 1 >= 0:
                    out_h[c - 1].wait()
                gather_h[c + 1] = start_gather(c + 1)
            gather_h[c].wait()
            out_h[c] = start_out(c)
        for c in range(max(0, nchunk - 2), nchunk):
            out_h[c].wait()

    return functools.partial(
        pl.kernel,
        mesh=plsc.VectorSubcoreMesh(core_axis_name="c", subcore_axis_name="s"),
        out_type=jax.ShapeDtypeStruct((n_tok, HIDDEN), jnp.float32),
        scratch_types=[
            pltpu.VMEM((per_worker,), jnp.int32),
            pltpu.VMEM((CHUNK, HIDDEN), jnp.float32),
            pltpu.VMEM((CHUNK, HIDDEN), jnp.float32),
            pltpu.SemaphoreType.DMA,
            pltpu.SemaphoreType.DMA,
            pltpu.SemaphoreType.DMA,
            pltpu.SemaphoreType.DMA,
        ],
    )(body)


PIECES = (16, 16)


def _ln_block(gath_ref, pp_ref, tf_ref, diff_ref, g_ref, b_ref):
    x = (gath_ref[0] + pp_ref[...]
         + tf_ref[0, 0, :][:, None] * diff_ref[0][None, :])
    mean = jnp.mean(x, axis=-1, keepdims=True)
    var = jnp.mean(x * x, axis=-1, keepdims=True) - mean * mean
    inv = lax.rsqrt(var + EPS)
    return (x - mean) * inv * g_ref[0][None, :] + b_ref[0][None, :]


def _tc_first_body(gath_ref, pp_ref, tf_ref, diff_ref, g_ref, b_ref, o_ref):
    o_ref[0] = _ln_block(gath_ref, pp_ref, tf_ref, diff_ref, g_ref, b_ref)


def _tc_next_body(prev_ref, gath_ref, pp_ref, tf_ref, diff_ref, g_ref,
                  b_ref, o_ref):
    del prev_ref
    o_ref[0] = _ln_block(gath_ref, pp_ref, tf_ref, diff_ref, g_ref, b_ref)


_SMALL_SPECS = [
    pl.BlockSpec((S, HIDDEN), lambda b: (0, 0)),
    pl.BlockSpec((1, 1, S), lambda b: (b, 0, 0)),
    pl.BlockSpec((1, HIDDEN), lambda b: (0, 0)),
    pl.BlockSpec((1, HIDDEN), lambda b: (0, 0)),
    pl.BlockSpec((1, HIDDEN), lambda b: (0, 0)),
]


def _make_tc_ln(n_rows, row_off):
    first = row_off == 0
    body = _tc_first_body if first else _tc_next_body
    gspec = [pl.BlockSpec((1, S, HIDDEN), lambda b: (b, 0, 0))]
    if not first:
        gspec = [pl.BlockSpec(memory_space=pl.ANY)] + gspec
    return pl.pallas_call(
        body,
        grid=(n_rows,),
        in_specs=gspec + _SMALL_SPECS,
        out_specs=pl.BlockSpec(
            (1, S, HIDDEN), lambda b: (b + row_off, 0, 0)),
        out_shape=jax.ShapeDtypeStruct((B, S, HIDDEN), jnp.float32),
        input_output_aliases={} if first else {0: 0},
    )


_sc_gathers = [_make_sc_gather(n) for n in PIECES]
_tc_lns = []
_off = 0
for _n in PIECES:
    _tc_lns.append(_make_tc_ln(_n, _off))
    _off += _n


def kernel(input_ids, token_type_ids, word_table, pos_table, type_table,
           name_table, gamma, beta):
    ids = input_ids.astype(jnp.int32).reshape(B * S)
    pos_plus = pos_table + name_table[0][None, :] + type_table[0][None, :]
    diff = (type_table[1] - type_table[0])[None, :]
    tf = token_type_ids.astype(jnp.float32).reshape(B, 1, S)
    gamma2 = gamma[None, :]
    beta2 = beta[None, :]

    gs = []
    off = 0
    for i, n in enumerate(PIECES):
        gs.append(_sc_gathers[i](
            ids[off * S:(off + n) * S], word_table).reshape(n, S, HIDDEN))
        off += n

    out = None
    off = 0
    for i, n in enumerate(PIECES):
        args = (gs[i], pos_plus, tf[off:off + n], diff, gamma2, beta2)
        if i == 0:
            out = _tc_lns[i](*args)
        else:
            out = _tc_lns[i](out, *args)
        off += n
    return out
